# baseline (speedup 1.0000x reference)
import functools
import math

import jax
import jax.numpy as jnp
from jax import lax
from jax.experimental import pallas as pl
from jax.experimental.pallas import tpu as pltpu
from jax.experimental.pallas import tpu_sc as plsc

D_MODEL = 64
SCALE = math.sqrt(D_MODEL)

_NC = 2
_NS = 16
_NW = _NC * _NS
_CHUNK = 128
_LANES = 16


@functools.lru_cache(maxsize=None)
def _make_sc_kernel(B):
    assert B % (_NW * _CHUNK) == 0
    rows_per_w = B // _NW
    nch = rows_per_w // _CHUNK

    mesh = plsc.VectorSubcoreMesh(core_axis_name="c", subcore_axis_name="s")

    @functools.partial(
        pl.kernel,
        mesh=mesh,
        out_type=jax.ShapeDtypeStruct((B // 2, 2 * D_MODEL), jnp.float32),
        scratch_types=[
            pltpu.VMEM((nch, _CHUNK), jnp.int32),
            pltpu.VMEM((_CHUNK,), jnp.int32),
            pltpu.VMEM((_CHUNK, 2 * D_MODEL), jnp.float32),
            pltpu.VMEM((_CHUNK // 2, 2 * D_MODEL), jnp.float32),
            pltpu.SemaphoreType.DMA,
        ],
    )
    def k(x_hbm, t2_hbm, out_hbm, idx_v, pair_v, in_v, out_v, sem):
        wid = lax.axis_index("s") * _NC + lax.axis_index("c")
        base_idx_row = wid * nch
        base_out = wid * (rows_per_w // 2)
        pltpu.sync_copy(x_hbm.at[pl.ds(base_idx_row, nch)], idx_v)

        def chunk_body(j, carry):
            def pair_body(kk, c):
                sl = pl.ds(kk * _LANES, _LANES)
                pair_v[sl] = lax.shift_right_logical(idx_v[j, sl], 1)
                return c

            lax.fori_loop(0, _CHUNK // _LANES, pair_body, 0)
            pltpu.async_copy(t2_hbm.at[pair_v], in_v, sem).wait()

            def group_body(g, c):
                idxv = idx_v[j, pl.ds(g * _LANES, _LANES)]
                base = (idxv & 1) * D_MODEL
                for ll in range(_LANES):
                    b = base[ll]
                    for kk in range(D_MODEL // _LANES):
                        o = kk * _LANES
                        out_v[
                            g * (_LANES // 2) + ll // 2,
                            pl.ds((ll % 2) * D_MODEL + o, _LANES),
                        ] = in_v[g * _LANES + ll, pl.ds(b + o, _LANES)] * SCALE
                return c

            lax.fori_loop(0, _CHUNK // _LANES, group_body, 0)
            pltpu.sync_copy(
                out_v,
                out_hbm.at[pl.ds(base_out + j * (_CHUNK // 2), _CHUNK // 2)],
            )
            return carry

        lax.fori_loop(0, nch, chunk_body, 0)

    return k


def kernel(x, table):
    B = x.size
    x2 = x.reshape(-1, _CHUNK).astype(jnp.int32)
    t2 = table.reshape(-1, 2 * D_MODEL)
    out = _make_sc_kernel(B)(x2, t2)
    return out.reshape(x.shape + (D_MODEL,))
